# baseline (speedup 1.0000x reference)
import functools
import math

import jax
import jax.numpy as jnp
from jax.experimental import pallas as pl


_N_EXP = 8
_TOP_K = 2
_CAP_FACTOR = 1.25
_MIN_CAP = 4


def _routing_kernel(x_ref, wg_ref, route_ref, uc_ref, *, capacity):
    x = x_ref[:]
    wg = wg_ref[:]
    logits = jax.lax.dot_general(
        x, wg, (((1,), (1,)), ((), ())),
        preferred_element_type=jnp.float32)
    T, E = logits.shape
    lane = jax.lax.broadcasted_iota(jnp.int32, (T, E), 1)

    m0 = jnp.max(logits, axis=1, keepdims=True)
    idx0 = jnp.min(jnp.where(logits == m0, lane, E), axis=1,
                   keepdims=True)
    masked = jnp.where(lane == idx0, -jnp.inf, logits)
    m1 = jnp.max(masked, axis=1, keepdims=True)
    idx1 = jnp.min(jnp.where(masked == m1, lane, E), axis=1,
                   keepdims=True)

    s = jnp.exp(m1 - m0)
    denom = 1.0 + s
    p0 = 1.0 / denom
    p1 = s / denom

    cnt0 = (lane == idx0).astype(jnp.float32)
    cnt1 = (lane == idx1).astype(jnp.float32)

    cnt = jnp.concatenate([cnt0, cnt1], axis=1).astype(jnp.bfloat16)
    row = jax.lax.broadcasted_iota(jnp.int32, (T, T), 0)
    col = jax.lax.broadcasted_iota(jnp.int32, (T, T), 1)
    tri = (row >= col).astype(jnp.bfloat16)
    csum = jax.lax.dot_general(
        tri, cnt, (((1,), (0,)), ((), ())),
        preferred_element_type=jnp.float32)
    csum0 = csum[:, :E]
    csum1 = csum[:, E:]
    total0 = csum0[T - 1:T, :]
    rank0_full = csum0 - cnt0
    rank1_full = total0 + csum1 - cnt1
    r0 = jnp.sum(rank0_full * cnt0, axis=1, keepdims=True)
    r1 = jnp.sum(rank1_full * cnt1, axis=1, keepdims=True)

    keep0 = (r0 < capacity).astype(jnp.float32)
    keep1 = (r1 < capacity).astype(jnp.float32)
    uc_ref[:] = jnp.sum(cnt0 * keep0 + cnt1 * keep1, axis=0, keepdims=True)

    zeros = jnp.zeros_like(p0)
    route_ref[:] = jnp.concatenate(
        [idx0.astype(jnp.float32), idx1.astype(jnp.float32),
         p0, p1, r0, r1, zeros, zeros], axis=1)


def _dispatch_kernel_f32(route_ref, cb_ref, mask_ref, *, capacity):
    r = route_ref[:]
    Tb = r.shape[0]
    idx0 = r[:, 0:1].astype(jnp.int32)
    idx1 = r[:, 1:2].astype(jnp.int32)
    p0 = r[:, 2:3]
    p1 = r[:, 3:4]
    r0 = r[:, 4:5].astype(jnp.int32)
    r1 = r[:, 5:6].astype(jnp.int32)
    t0 = jnp.where(r0 < capacity, idx0 * capacity + r0, -1)
    t1 = jnp.where(r1 < capacity, idx1 * capacity + r1, -1)
    col = jax.lax.broadcasted_iota(jnp.int32, (Tb, _N_EXP * capacity), 1)
    v0 = jnp.where(col == t0, p0, 0.0)
    v1 = jnp.where(col == t1, p1, 0.0)
    out = v0 + v1
    cb_ref[:] = out
    mask_ref[:] = out != 0.0


def _dispatch_kernel(route_ref, cb_ref, mask_ref, *, capacity):
    r = route_ref[:]
    Tb = r.shape[0]
    idx0 = r[:, 0:1].reshape(Tb, 1, 1).astype(jnp.int32)
    idx1 = r[:, 1:2].reshape(Tb, 1, 1).astype(jnp.int32)
    p0 = r[:, 2:3].reshape(Tb, 1, 1)
    p1 = r[:, 3:4].reshape(Tb, 1, 1)
    r0 = r[:, 4:5].reshape(Tb, 1, 1).astype(jnp.int32)
    r1 = r[:, 5:6].reshape(Tb, 1, 1).astype(jnp.int32)
    shp = (Tb, _N_EXP, capacity)
    e_io = jax.lax.broadcasted_iota(jnp.int32, shp, 1)
    c_io = jax.lax.broadcasted_iota(jnp.int32, shp, 2)
    v0 = jnp.where((e_io == idx0) & (c_io == r0), p0, 0.0)
    v1 = jnp.where((e_io == idx1) & (c_io == r1), p1, 0.0)
    out = v0 + v1
    cb_ref[:] = out
    mask_ref[:] = out != 0.0


def kernel(x, w_g):
    Bx, Tx, H = x.shape
    num_tokens = Bx * Tx
    E = w_g.shape[0]
    capacity = int(max(math.floor(_TOP_K * _CAP_FACTOR * num_tokens / E),
                       _MIN_CAP))
    x_flat = x.reshape(num_tokens, H)

    route, uc = pl.pallas_call(
        functools.partial(_routing_kernel, capacity=capacity),
        out_shape=[
            jax.ShapeDtypeStruct((num_tokens, E), jnp.float32),
            jax.ShapeDtypeStruct((1, E), jnp.float32),
        ],
    )(x_flat, w_g)
    used_capacity = uc.reshape(E).astype(jnp.int32)

    Tb = 128
    cb, mask = pl.pallas_call(
        functools.partial(_dispatch_kernel_f32, capacity=capacity),
        grid=(num_tokens // Tb,),
        in_specs=[pl.BlockSpec((Tb, E), lambda i: (i, 0))],
        out_specs=[
            pl.BlockSpec((Tb, E * capacity), lambda i: (i, 0)),
            pl.BlockSpec((Tb, E * capacity), lambda i: (i, 0)),
        ],
        out_shape=[
            jax.ShapeDtypeStruct((num_tokens, E * capacity), jnp.float32),
            jax.ShapeDtypeStruct((num_tokens, E * capacity), jnp.bool_),
        ],
    )(route)
    cb = cb.reshape(num_tokens, E, capacity)
    mask = mask.reshape(num_tokens, E, capacity)
    return (used_capacity, cb, mask)


# speedup vs baseline: 1.4578x; 1.4578x over previous
import functools
import math

import jax
import jax.numpy as jnp
from jax.experimental import pallas as pl


_N_EXP = 8
_TOP_K = 2
_CAP_FACTOR = 1.25
_MIN_CAP = 4


def _routing_kernel(x_ref, wg_ref, route_ref, uc_ref, *, capacity):
    x = x_ref[:]
    wg = wg_ref[:]
    logits = jax.lax.dot_general(
        x, wg, (((1,), (1,)), ((), ())),
        preferred_element_type=jnp.float32)
    T, E = logits.shape
    lane = jax.lax.broadcasted_iota(jnp.int32, (T, E), 1)

    m0 = jnp.max(logits, axis=1, keepdims=True)
    idx0 = jnp.min(jnp.where(logits == m0, lane, E), axis=1,
                   keepdims=True)
    masked = jnp.where(lane == idx0, -jnp.inf, logits)
    m1 = jnp.max(masked, axis=1, keepdims=True)
    idx1 = jnp.min(jnp.where(masked == m1, lane, E), axis=1,
                   keepdims=True)

    s = jnp.exp(m1 - m0)
    denom = 1.0 + s
    p0 = 1.0 / denom
    p1 = s / denom

    cnt0 = (lane == idx0).astype(jnp.float32)
    cnt1 = (lane == idx1).astype(jnp.float32)

    cnt = jnp.concatenate([cnt0, cnt1], axis=1).astype(jnp.bfloat16)
    row = jax.lax.broadcasted_iota(jnp.int32, (T, T), 0)
    col = jax.lax.broadcasted_iota(jnp.int32, (T, T), 1)
    tri = (row >= col).astype(jnp.bfloat16)
    csum = jax.lax.dot_general(
        tri, cnt, (((1,), (0,)), ((), ())),
        preferred_element_type=jnp.float32)
    csum0 = csum[:, :E]
    csum1 = csum[:, E:]
    total0 = csum0[T - 1:T, :]
    rank0_full = csum0 - cnt0
    rank1_full = total0 + csum1 - cnt1
    r0 = jnp.sum(rank0_full * cnt0, axis=1, keepdims=True)
    r1 = jnp.sum(rank1_full * cnt1, axis=1, keepdims=True)

    keep0 = (r0 < capacity).astype(jnp.float32)
    keep1 = (r1 < capacity).astype(jnp.float32)
    uc_ref[:] = jnp.sum(cnt0 * keep0 + cnt1 * keep1, axis=0, keepdims=True)

    zeros = jnp.zeros_like(p0)
    route_ref[:] = jnp.concatenate(
        [idx0.astype(jnp.float32), idx1.astype(jnp.float32),
         p0, p1, r0, r1, zeros, zeros], axis=1)


def _dispatch_kernel_f32(route_ref, cb_ref, mask_ref, *, capacity):
    r = route_ref[:]
    Tb = r.shape[0]
    idx0 = r[:, 0:1].astype(jnp.int32)
    idx1 = r[:, 1:2].astype(jnp.int32)
    p0 = r[:, 2:3]
    p1 = r[:, 3:4]
    r0 = r[:, 4:5].astype(jnp.int32)
    r1 = r[:, 5:6].astype(jnp.int32)
    col = jax.lax.broadcasted_iota(jnp.int32, (Tb, capacity), 1)
    for e in range(_N_EXP):
        v0 = jnp.where((idx0 == e) & (col == r0), p0, 0.0)
        v1 = jnp.where((idx1 == e) & (col == r1), p1, 0.0)
        out = v0 + v1
        cb_ref[:, e, :] = out
        mask_ref[:, e, :] = out != 0.0


def _dispatch_kernel(route_ref, cb_ref, mask_ref, *, capacity):
    r = route_ref[:]
    Tb = r.shape[0]
    idx0 = r[:, 0:1].reshape(Tb, 1, 1).astype(jnp.int32)
    idx1 = r[:, 1:2].reshape(Tb, 1, 1).astype(jnp.int32)
    p0 = r[:, 2:3].reshape(Tb, 1, 1)
    p1 = r[:, 3:4].reshape(Tb, 1, 1)
    r0 = r[:, 4:5].reshape(Tb, 1, 1).astype(jnp.int32)
    r1 = r[:, 5:6].reshape(Tb, 1, 1).astype(jnp.int32)
    shp = (Tb, _N_EXP, capacity)
    e_io = jax.lax.broadcasted_iota(jnp.int32, shp, 1)
    c_io = jax.lax.broadcasted_iota(jnp.int32, shp, 2)
    v0 = jnp.where((e_io == idx0) & (c_io == r0), p0, 0.0)
    v1 = jnp.where((e_io == idx1) & (c_io == r1), p1, 0.0)
    out = v0 + v1
    cb_ref[:] = out
    mask_ref[:] = out != 0.0


def kernel(x, w_g):
    Bx, Tx, H = x.shape
    num_tokens = Bx * Tx
    E = w_g.shape[0]
    capacity = int(max(math.floor(_TOP_K * _CAP_FACTOR * num_tokens / E),
                       _MIN_CAP))
    x_flat = x.reshape(num_tokens, H)

    route, uc = pl.pallas_call(
        functools.partial(_routing_kernel, capacity=capacity),
        out_shape=[
            jax.ShapeDtypeStruct((num_tokens, E), jnp.float32),
            jax.ShapeDtypeStruct((1, E), jnp.float32),
        ],
    )(x_flat, w_g)
    used_capacity = uc.reshape(E).astype(jnp.int32)

    Tb = 128
    cb, mask = pl.pallas_call(
        functools.partial(_dispatch_kernel_f32, capacity=capacity),
        grid=(num_tokens // Tb,),
        in_specs=[pl.BlockSpec((Tb, E), lambda i: (i, 0))],
        out_specs=[
            pl.BlockSpec((Tb, E, capacity), lambda i: (i, 0, 0)),
            pl.BlockSpec((Tb, E, capacity), lambda i: (i, 0, 0)),
        ],
        out_shape=[
            jax.ShapeDtypeStruct((num_tokens, E, capacity), jnp.float32),
            jax.ShapeDtypeStruct((num_tokens, E, capacity), jnp.bool_),
        ],
    )(route)
    return (used_capacity, cb, mask)


# speedup vs baseline: 1.5264x; 1.0471x over previous
import functools
import math

import jax
import jax.numpy as jnp
from jax.experimental import pallas as pl


_N_EXP = 8
_TOP_K = 2
_CAP_FACTOR = 1.25
_MIN_CAP = 4


def _routing_kernel(x_ref, wg_ref, route_ref, uc_ref, *, capacity):
    x = x_ref[:]
    wg = wg_ref[:]
    logits = jax.lax.dot_general(
        x, wg, (((1,), (1,)), ((), ())),
        preferred_element_type=jnp.float32)
    T, E = logits.shape
    lane = jax.lax.broadcasted_iota(jnp.int32, (T, E), 1)

    m0 = jnp.max(logits, axis=1, keepdims=True)
    idx0 = jnp.min(jnp.where(logits == m0, lane, E), axis=1,
                   keepdims=True)
    masked = jnp.where(lane == idx0, -jnp.inf, logits)
    m1 = jnp.max(masked, axis=1, keepdims=True)
    idx1 = jnp.min(jnp.where(masked == m1, lane, E), axis=1,
                   keepdims=True)

    s = jnp.exp(m1 - m0)
    denom = 1.0 + s
    p0 = 1.0 / denom
    p1 = s / denom

    cnt0 = (lane == idx0).astype(jnp.float32)
    cnt1 = (lane == idx1).astype(jnp.float32)

    cnt = jnp.concatenate([cnt0, cnt1], axis=1).astype(jnp.bfloat16)
    row = jax.lax.broadcasted_iota(jnp.int32, (T, T), 0)
    col = jax.lax.broadcasted_iota(jnp.int32, (T, T), 1)
    tri = (row >= col).astype(jnp.bfloat16)
    csum = jax.lax.dot_general(
        tri, cnt, (((1,), (0,)), ((), ())),
        preferred_element_type=jnp.float32)
    csum0 = csum[:, :E]
    csum1 = csum[:, E:]
    total0 = csum0[T - 1:T, :]
    rank0_full = csum0 - cnt0
    rank1_full = total0 + csum1 - cnt1
    r0 = jnp.sum(rank0_full * cnt0, axis=1, keepdims=True)
    r1 = jnp.sum(rank1_full * cnt1, axis=1, keepdims=True)

    keep0 = (r0 < capacity).astype(jnp.float32)
    keep1 = (r1 < capacity).astype(jnp.float32)
    uc_ref[:] = jnp.sum(cnt0 * keep0 + cnt1 * keep1, axis=0, keepdims=True)

    zeros = jnp.zeros_like(p0)
    route_ref[:] = jnp.concatenate(
        [idx0.astype(jnp.float32), idx1.astype(jnp.float32),
         p0, p1, r0, r1, zeros, zeros], axis=1)


def _dispatch_kernel_f32(route_ref, cb_ref, mask_ref, *, capacity):
    r = route_ref[:]
    Tb = r.shape[0]
    idx0 = r[:, 0:1].astype(jnp.int32)
    idx1 = r[:, 1:2].astype(jnp.int32)
    p0 = r[:, 2:3]
    p1 = r[:, 3:4]
    r0 = r[:, 4:5].astype(jnp.int32)
    r1 = r[:, 5:6].astype(jnp.int32)
    col = jax.lax.broadcasted_iota(jnp.int32, (Tb, capacity), 1)
    for e in range(_N_EXP):
        hit0 = idx0 == e
        hit1 = idx1 == e
        re = jnp.where(hit0, r0, jnp.where(hit1, r1, -1))
        pe = jnp.where(hit0, p0, p1)
        cmp = col == re
        cb_ref[:, e, :] = jnp.where(cmp, pe, 0.0)
        mask_ref[:, e, :] = cmp & (pe != 0.0)


def _dispatch_kernel(route_ref, cb_ref, mask_ref, *, capacity):
    r = route_ref[:]
    Tb = r.shape[0]
    idx0 = r[:, 0:1].reshape(Tb, 1, 1).astype(jnp.int32)
    idx1 = r[:, 1:2].reshape(Tb, 1, 1).astype(jnp.int32)
    p0 = r[:, 2:3].reshape(Tb, 1, 1)
    p1 = r[:, 3:4].reshape(Tb, 1, 1)
    r0 = r[:, 4:5].reshape(Tb, 1, 1).astype(jnp.int32)
    r1 = r[:, 5:6].reshape(Tb, 1, 1).astype(jnp.int32)
    shp = (Tb, _N_EXP, capacity)
    e_io = jax.lax.broadcasted_iota(jnp.int32, shp, 1)
    c_io = jax.lax.broadcasted_iota(jnp.int32, shp, 2)
    v0 = jnp.where((e_io == idx0) & (c_io == r0), p0, 0.0)
    v1 = jnp.where((e_io == idx1) & (c_io == r1), p1, 0.0)
    out = v0 + v1
    cb_ref[:] = out
    mask_ref[:] = out != 0.0


def kernel(x, w_g):
    Bx, Tx, H = x.shape
    num_tokens = Bx * Tx
    E = w_g.shape[0]
    capacity = int(max(math.floor(_TOP_K * _CAP_FACTOR * num_tokens / E),
                       _MIN_CAP))
    x_flat = x.reshape(num_tokens, H)

    route, uc = pl.pallas_call(
        functools.partial(_routing_kernel, capacity=capacity),
        out_shape=[
            jax.ShapeDtypeStruct((num_tokens, E), jnp.float32),
            jax.ShapeDtypeStruct((1, E), jnp.float32),
        ],
    )(x_flat, w_g)
    used_capacity = uc.reshape(E).astype(jnp.int32)

    Tb = 128
    cb, mask = pl.pallas_call(
        functools.partial(_dispatch_kernel_f32, capacity=capacity),
        grid=(num_tokens // Tb,),
        in_specs=[pl.BlockSpec((Tb, E), lambda i: (i, 0))],
        out_specs=[
            pl.BlockSpec((Tb, E, capacity), lambda i: (i, 0, 0)),
            pl.BlockSpec((Tb, E, capacity), lambda i: (i, 0, 0)),
        ],
        out_shape=[
            jax.ShapeDtypeStruct((num_tokens, E, capacity), jnp.float32),
            jax.ShapeDtypeStruct((num_tokens, E, capacity), jnp.bool_),
        ],
    )(route)
    return (used_capacity, cb, mask)


# speedup vs baseline: 1.5820x; 1.0364x over previous
import functools
import math

import jax
import jax.numpy as jnp
from jax.experimental import pallas as pl
from jax.experimental.pallas import tpu as pltpu


_N_EXP = 8
_TOP_K = 2
_CAP_FACTOR = 1.25
_MIN_CAP = 4


def _routing_kernel(x_ref, wg_ref, route_ref, uc_ref, logits_ref, *,
                    capacity):
    i = pl.program_id(0)
    n = pl.num_programs(0)
    chunk = x_ref.shape[0]
    logits_ref[pl.ds(i * chunk, chunk), :] = jax.lax.dot_general(
        x_ref[:], wg_ref[:], (((1,), (1,)), ((), ())),
        preferred_element_type=jnp.float32)

    @pl.when(i == n - 1)
    def _finish():
        _routing_finish(logits_ref, route_ref, uc_ref, capacity)


def _routing_finish(logits_ref, route_ref, uc_ref, capacity):
    logits = logits_ref[:]
    T, E = logits.shape
    lane = jax.lax.broadcasted_iota(jnp.int32, (T, E), 1)

    m0 = jnp.max(logits, axis=1, keepdims=True)
    idx0 = jnp.min(jnp.where(logits == m0, lane, E), axis=1,
                   keepdims=True)
    masked = jnp.where(lane == idx0, -jnp.inf, logits)
    m1 = jnp.max(masked, axis=1, keepdims=True)
    idx1 = jnp.min(jnp.where(masked == m1, lane, E), axis=1,
                   keepdims=True)

    s = jnp.exp(m1 - m0)
    denom = 1.0 + s
    p0 = 1.0 / denom
    p1 = s / denom

    cnt0 = (lane == idx0).astype(jnp.float32)
    cnt1 = (lane == idx1).astype(jnp.float32)

    cnt = jnp.concatenate([cnt0, cnt1], axis=1).astype(jnp.bfloat16)
    row = jax.lax.broadcasted_iota(jnp.int32, (T, T), 0)
    col = jax.lax.broadcasted_iota(jnp.int32, (T, T), 1)
    tri = (row >= col).astype(jnp.bfloat16)
    csum = jax.lax.dot_general(
        tri, cnt, (((1,), (0,)), ((), ())),
        preferred_element_type=jnp.float32)
    csum0 = csum[:, :E]
    csum1 = csum[:, E:]
    total0 = csum0[T - 1:T, :]
    rank0_full = csum0 - cnt0
    rank1_full = total0 + csum1 - cnt1
    r0 = jnp.sum(rank0_full * cnt0, axis=1, keepdims=True)
    r1 = jnp.sum(rank1_full * cnt1, axis=1, keepdims=True)

    keep0 = (r0 < capacity).astype(jnp.float32)
    keep1 = (r1 < capacity).astype(jnp.float32)
    uc_ref[:] = jnp.sum(cnt0 * keep0 + cnt1 * keep1, axis=0, keepdims=True)

    zeros = jnp.zeros_like(p0)
    route_ref[:] = jnp.concatenate(
        [idx0.astype(jnp.float32), idx1.astype(jnp.float32),
         p0, p1, r0, r1, zeros, zeros], axis=1)


def _dispatch_kernel_f32(route_ref, cb_ref, mask_ref, *, capacity):
    r = route_ref[:]
    Tb = r.shape[0]
    idx0 = r[:, 0:1].astype(jnp.int32)
    idx1 = r[:, 1:2].astype(jnp.int32)
    p0 = r[:, 2:3]
    p1 = r[:, 3:4]
    r0 = r[:, 4:5].astype(jnp.int32)
    r1 = r[:, 5:6].astype(jnp.int32)
    col = jax.lax.broadcasted_iota(jnp.int32, (Tb, capacity), 1)
    for e in range(_N_EXP):
        hit0 = idx0 == e
        hit1 = idx1 == e
        re = jnp.where(hit0, r0, jnp.where(hit1, r1, -1))
        pe = jnp.where(hit0, p0, p1)
        cmp = col == re
        cb_ref[:, e, :] = jnp.where(cmp, pe, 0.0)
        mask_ref[:, e, :] = cmp & (pe != 0.0)


def _dispatch_kernel(route_ref, cb_ref, mask_ref, *, capacity):
    r = route_ref[:]
    Tb = r.shape[0]
    idx0 = r[:, 0:1].reshape(Tb, 1, 1).astype(jnp.int32)
    idx1 = r[:, 1:2].reshape(Tb, 1, 1).astype(jnp.int32)
    p0 = r[:, 2:3].reshape(Tb, 1, 1)
    p1 = r[:, 3:4].reshape(Tb, 1, 1)
    r0 = r[:, 4:5].reshape(Tb, 1, 1).astype(jnp.int32)
    r1 = r[:, 5:6].reshape(Tb, 1, 1).astype(jnp.int32)
    shp = (Tb, _N_EXP, capacity)
    e_io = jax.lax.broadcasted_iota(jnp.int32, shp, 1)
    c_io = jax.lax.broadcasted_iota(jnp.int32, shp, 2)
    v0 = jnp.where((e_io == idx0) & (c_io == r0), p0, 0.0)
    v1 = jnp.where((e_io == idx1) & (c_io == r1), p1, 0.0)
    out = v0 + v1
    cb_ref[:] = out
    mask_ref[:] = out != 0.0


def kernel(x, w_g):
    Bx, Tx, H = x.shape
    num_tokens = Bx * Tx
    E = w_g.shape[0]
    capacity = int(max(math.floor(_TOP_K * _CAP_FACTOR * num_tokens / E),
                       _MIN_CAP))
    x_flat = x.reshape(num_tokens, H)

    Tc = 256
    route, uc = pl.pallas_call(
        functools.partial(_routing_kernel, capacity=capacity),
        grid=(num_tokens // Tc,),
        in_specs=[
            pl.BlockSpec((Tc, H), lambda i: (i, 0)),
            pl.BlockSpec((E, H), lambda i: (0, 0)),
        ],
        out_specs=[
            pl.BlockSpec((num_tokens, E), lambda i: (0, 0)),
            pl.BlockSpec((1, E), lambda i: (0, 0)),
        ],
        out_shape=[
            jax.ShapeDtypeStruct((num_tokens, E), jnp.float32),
            jax.ShapeDtypeStruct((1, E), jnp.float32),
        ],
        scratch_shapes=[pltpu.VMEM((num_tokens, E), jnp.float32)],
    )(x_flat, w_g)
    used_capacity = uc.reshape(E).astype(jnp.int32)

    Tb = 128
    cb, mask = pl.pallas_call(
        functools.partial(_dispatch_kernel_f32, capacity=capacity),
        grid=(num_tokens // Tb,),
        in_specs=[pl.BlockSpec((Tb, E), lambda i: (i, 0))],
        out_specs=[
            pl.BlockSpec((Tb, E, capacity), lambda i: (i, 0, 0)),
            pl.BlockSpec((Tb, E, capacity), lambda i: (i, 0, 0)),
        ],
        out_shape=[
            jax.ShapeDtypeStruct((num_tokens, E, capacity), jnp.float32),
            jax.ShapeDtypeStruct((num_tokens, E, capacity), jnp.bool_),
        ],
    )(route)
    return (used_capacity, cb, mask)


# speedup vs baseline: 1.6247x; 1.0270x over previous
import functools
import math

import jax
import jax.numpy as jnp
from jax.experimental import pallas as pl
from jax.experimental.pallas import tpu as pltpu


_N_EXP = 8
_TOP_K = 2
_CAP_FACTOR = 1.25
_MIN_CAP = 4


def _routing_finish(logits_ref, route_ref, uc_ref, capacity):
    logits = logits_ref[:]
    T, E = logits.shape
    lane = jax.lax.broadcasted_iota(jnp.int32, (T, E), 1)

    m0 = jnp.max(logits, axis=1, keepdims=True)
    idx0 = jnp.min(jnp.where(logits == m0, lane, E), axis=1,
                   keepdims=True)
    masked = jnp.where(lane == idx0, -jnp.inf, logits)
    m1 = jnp.max(masked, axis=1, keepdims=True)
    idx1 = jnp.min(jnp.where(masked == m1, lane, E), axis=1,
                   keepdims=True)

    s = jnp.exp(m1 - m0)
    denom = 1.0 + s
    p0 = 1.0 / denom
    p1 = s / denom

    cnt0 = (lane == idx0).astype(jnp.float32)
    cnt1 = (lane == idx1).astype(jnp.float32)

    cnt = jnp.concatenate([cnt0, cnt1], axis=1).astype(jnp.bfloat16)
    row = jax.lax.broadcasted_iota(jnp.int32, (T, T), 0)
    col = jax.lax.broadcasted_iota(jnp.int32, (T, T), 1)
    tri = (row >= col).astype(jnp.bfloat16)
    csum = jax.lax.dot_general(
        tri, cnt, (((1,), (0,)), ((), ())),
        preferred_element_type=jnp.float32)
    csum0 = csum[:, :E]
    csum1 = csum[:, E:]
    total0 = csum0[T - 1:T, :]
    rank0_full = csum0 - cnt0
    rank1_full = total0 + csum1 - cnt1
    r0 = jnp.sum(rank0_full * cnt0, axis=1, keepdims=True)
    r1 = jnp.sum(rank1_full * cnt1, axis=1, keepdims=True)

    keep0 = (r0 < capacity).astype(jnp.float32)
    keep1 = (r1 < capacity).astype(jnp.float32)
    uc_ref[:] = jnp.sum(cnt0 * keep0 + cnt1 * keep1, axis=0, keepdims=True)

    zeros = jnp.zeros_like(p0)
    route_ref[:] = jnp.concatenate(
        [idx0.astype(jnp.float32), idx1.astype(jnp.float32),
         p0, p1, r0, r1, zeros, zeros], axis=1)


def _fused_kernel(x_ref, wg_ref, uc_ref, cb_ref, mask_ref,
                  logits_ref, route_ref, *, capacity, n_route, dispatch_tb):
    i = pl.program_id(0)

    @pl.when(i < n_route)
    def _route_step():
        chunk = x_ref.shape[0]
        logits_ref[pl.ds(i * chunk, chunk), :] = jax.lax.dot_general(
            x_ref[:], wg_ref[:], (((1,), (1,)), ((), ())),
            preferred_element_type=jnp.float32)

    @pl.when(i == n_route - 1)
    def _route_finish():
        _routing_finish(logits_ref, route_ref, uc_ref, capacity)

    @pl.when(i >= n_route)
    def _dispatch_step():
        j = i - n_route
        r = route_ref[pl.ds(j * dispatch_tb, dispatch_tb), :]
        Tb = dispatch_tb
        idx0 = r[:, 0:1].astype(jnp.int32)
        idx1 = r[:, 1:2].astype(jnp.int32)
        p0 = r[:, 2:3]
        p1 = r[:, 3:4]
        r0 = r[:, 4:5].astype(jnp.int32)
        r1 = r[:, 5:6].astype(jnp.int32)
        col = jax.lax.broadcasted_iota(jnp.int32, (Tb, capacity), 1)
        for e in range(_N_EXP):
            hit0 = idx0 == e
            hit1 = idx1 == e
            re = jnp.where(hit0, r0, jnp.where(hit1, r1, -1))
            pe = jnp.where(hit0, p0, p1)
            re = jnp.where(pe != 0.0, re, -1)
            cmp = col == re
            cb_ref[:, e, :] = jnp.where(cmp, pe, 0.0)
            mask_ref[:, e, :] = cmp


def kernel(x, w_g):
    Bx, Tx, H = x.shape
    num_tokens = Bx * Tx
    E = w_g.shape[0]
    capacity = int(max(math.floor(_TOP_K * _CAP_FACTOR * num_tokens / E),
                       _MIN_CAP))
    x_flat = x.reshape(num_tokens, H)

    Tc = 256
    Tb = 128
    n_route = num_tokens // Tc
    n_disp = num_tokens // Tb

    uc, cb, mask = pl.pallas_call(
        functools.partial(_fused_kernel, capacity=capacity,
                          n_route=n_route, dispatch_tb=Tb),
        grid=(n_route + n_disp,),
        in_specs=[
            pl.BlockSpec((Tc, H),
                         lambda i: (jnp.minimum(i, n_route - 1), 0)),
            pl.BlockSpec((E, H), lambda i: (0, 0)),
        ],
        out_specs=[
            pl.BlockSpec((1, E), lambda i: (0, 0)),
            pl.BlockSpec((Tb, E, capacity),
                         lambda i: (jnp.maximum(i - n_route, 0), 0, 0)),
            pl.BlockSpec((Tb, E, capacity),
                         lambda i: (jnp.maximum(i - n_route, 0), 0, 0)),
        ],
        out_shape=[
            jax.ShapeDtypeStruct((1, E), jnp.float32),
            jax.ShapeDtypeStruct((num_tokens, E, capacity), jnp.float32),
            jax.ShapeDtypeStruct((num_tokens, E, capacity), jnp.bool_),
        ],
        scratch_shapes=[
            pltpu.VMEM((num_tokens, E), jnp.float32),
            pltpu.VMEM((num_tokens, E), jnp.float32),
        ],
    )(x_flat, w_g)
    used_capacity = uc.reshape(E).astype(jnp.int32)
    return (used_capacity, cb, mask)


# speedup vs baseline: 1.6779x; 1.0327x over previous
import functools
import math

import jax
import jax.numpy as jnp
from jax.experimental import pallas as pl
from jax.experimental.pallas import tpu as pltpu


_N_EXP = 8
_TOP_K = 2
_CAP_FACTOR = 1.25
_MIN_CAP = 4


def _routing_finish(logits_ref, route_ref, uc_ref, capacity):
    logits = logits_ref[:]
    T, E = logits.shape
    lane = jax.lax.broadcasted_iota(jnp.int32, (T, E), 1)

    m0 = jnp.max(logits, axis=1, keepdims=True)
    idx0 = jnp.min(jnp.where(logits == m0, lane, E), axis=1,
                   keepdims=True)
    masked = jnp.where(lane == idx0, -jnp.inf, logits)
    m1 = jnp.max(masked, axis=1, keepdims=True)
    idx1 = jnp.min(jnp.where(masked == m1, lane, E), axis=1,
                   keepdims=True)

    s = jnp.exp(m1 - m0)
    denom = 1.0 + s
    p0 = 1.0 / denom
    p1 = s / denom

    cnt0 = (lane == idx0).astype(jnp.float32)
    cnt1 = (lane == idx1).astype(jnp.float32)

    cnt = jnp.concatenate([cnt0, cnt1], axis=1).astype(jnp.bfloat16)
    row = jax.lax.broadcasted_iota(jnp.int32, (T, T), 0)
    col = jax.lax.broadcasted_iota(jnp.int32, (T, T), 1)
    tri = (row >= col).astype(jnp.bfloat16)
    csum = jax.lax.dot_general(
        tri, cnt, (((1,), (0,)), ((), ())),
        preferred_element_type=jnp.float32)
    csum0 = csum[:, :E]
    csum1 = csum[:, E:]
    total0 = csum0[T - 1:T, :]
    rank0_full = csum0 - cnt0
    rank1_full = total0 + csum1 - cnt1
    r0 = jnp.sum(rank0_full * cnt0, axis=1, keepdims=True)
    r1 = jnp.sum(rank1_full * cnt1, axis=1, keepdims=True)

    keep0 = (r0 < capacity).astype(jnp.float32)
    keep1 = (r1 < capacity).astype(jnp.float32)
    uc_ref[:] = jnp.sum(cnt0 * keep0 + cnt1 * keep1, axis=0, keepdims=True)

    zeros = jnp.zeros_like(p0)
    route_ref[:] = jnp.concatenate(
        [idx0.astype(jnp.float32), idx1.astype(jnp.float32),
         p0, p1, r0, r1, zeros, zeros], axis=1)


def _fused_kernel(x_ref, wg_ref, uc_ref, cb_ref, mask_ref,
                  logits_ref, route_ref, *, capacity, n_route, dispatch_tb):
    i = pl.program_id(0)

    @pl.when(i < n_route)
    def _route_step():
        chunk = x_ref.shape[0]
        logits_ref[pl.ds(i * chunk, chunk), :] = jax.lax.dot_general(
            x_ref[:], wg_ref[:], (((1,), (1,)), ((), ())),
            preferred_element_type=jnp.float32)

    @pl.when(i == n_route - 1)
    def _route_finish():
        _routing_finish(logits_ref, route_ref, uc_ref, capacity)

    @pl.when(i >= n_route)
    def _dispatch_step():
        j = i - n_route
        r = route_ref[pl.ds(j * dispatch_tb, dispatch_tb), :]
        Tb = dispatch_tb
        idx0 = r[:, 0:1].astype(jnp.int32)
        idx1 = r[:, 1:2].astype(jnp.int32)
        p0 = r[:, 2:3]
        p1 = r[:, 3:4]
        r0 = r[:, 4:5].astype(jnp.int32)
        r1 = r[:, 5:6].astype(jnp.int32)
        col = jax.lax.broadcasted_iota(jnp.int32, (Tb, capacity), 1)
        for e in range(_N_EXP):
            hit0 = idx0 == e
            hit1 = idx1 == e
            re = jnp.where(hit0, r0, jnp.where(hit1, r1, -1))
            pe = jnp.where(hit0, p0, p1)
            re = jnp.where(pe != 0.0, re, -1)
            cmp = col == re
            cb_ref[:, e, :] = jnp.where(cmp, pe, 0.0)
            mask_ref[:, e, :] = cmp


def kernel(x, w_g):
    Bx, Tx, H = x.shape
    num_tokens = Bx * Tx
    E = w_g.shape[0]
    capacity = int(max(math.floor(_TOP_K * _CAP_FACTOR * num_tokens / E),
                       _MIN_CAP))
    x_flat = x.reshape(num_tokens, H)

    Tc = 256
    Tb = 256
    n_route = num_tokens // Tc
    n_disp = num_tokens // Tb

    uc, cb, mask = pl.pallas_call(
        functools.partial(_fused_kernel, capacity=capacity,
                          n_route=n_route, dispatch_tb=Tb),
        grid=(n_route + n_disp,),
        in_specs=[
            pl.BlockSpec((Tc, H),
                         lambda i: (jnp.minimum(i, n_route - 1), 0)),
            pl.BlockSpec((E, H), lambda i: (0, 0)),
        ],
        out_specs=[
            pl.BlockSpec((1, E), lambda i: (0, 0)),
            pl.BlockSpec((Tb, E, capacity),
                         lambda i: (jnp.maximum(i - n_route, 0), 0, 0)),
            pl.BlockSpec((Tb, E, capacity),
                         lambda i: (jnp.maximum(i - n_route, 0), 0, 0)),
        ],
        out_shape=[
            jax.ShapeDtypeStruct((1, E), jnp.float32),
            jax.ShapeDtypeStruct((num_tokens, E, capacity), jnp.float32),
            jax.ShapeDtypeStruct((num_tokens, E, capacity), jnp.bool_),
        ],
        scratch_shapes=[
            pltpu.VMEM((num_tokens, E), jnp.float32),
            pltpu.VMEM((num_tokens, E), jnp.float32),
        ],
    )(x_flat, w_g)
    used_capacity = uc.reshape(E).astype(jnp.int32)
    return (used_capacity, cb, mask)
